# baseline (speedup 1.0000x reference)
import functools
import math

import jax
import jax.numpy as jnp
from jax.experimental import pallas as pl
from jax.experimental.pallas import tpu as pltpu

H = 8
TOPK = 8
EPS = 0.01


def _ln_stats(x):
    m = jnp.mean(x, axis=-1, keepdims=True)
    v = jnp.var(x, axis=-1, keepdims=True)
    return m, v


def _ln_apply(x, m, v, s, b):
    return (x - m) / jnp.sqrt(v + 1e-5) * s + b


def _sm_rows(s):
    s = s - jnp.max(s, axis=-1, keepdims=True)
    e = jnp.exp(s)
    return e / jnp.sum(e, axis=-1, keepdims=True)


def _heads_attn(q, k, v, hd, nreal, o_scr):
    nq = q.shape[0]
    for i in range(H):
        qh = q[:, i * hd:(i + 1) * hd]
        kh = k[:, i * hd:(i + 1) * hd]
        vh = v[:, i * hd:(i + 1) * hd]
        s = jax.lax.dot_general(qh, kh, (((1,), (1,)), ((), ())),
                                preferred_element_type=jnp.float32)
        s = s / jnp.sqrt(jnp.float32(hd))
        mask = jax.lax.broadcasted_iota(jnp.int32, s.shape, 1) < nreal
        p = _sm_rows(jnp.where(mask, s, -1e30))
        o_scr[:, i * hd:(i + 1) * hd] = jnp.dot(
            p, vh, preferred_element_type=jnp.float32)
    return o_scr[...]


def _ab1_kernel(nreal, hd, x_ref, m_ref, v_ref, s_ref, b_ref, wq_ref,
                wk_ref, wv_ref, wo_ref, o_ref, o_scr):
    x = x_ref[0]
    h = _ln_apply(x, m_ref[0], v_ref[0], s_ref[...], b_ref[...])
    q = jnp.dot(h, wq_ref[...], preferred_element_type=jnp.float32)
    k = jnp.dot(h, wk_ref[...], preferred_element_type=jnp.float32)
    v = jnp.dot(h, wv_ref[...], preferred_element_type=jnp.float32)
    o = _heads_attn(q, k, v, hd, nreal, o_scr)
    o_ref[0] = x + jnp.dot(o, wo_ref[...], preferred_element_type=jnp.float32)


def _attn_block1(x, m, v, s, b, wq, wk, wv, wo, nreal):
    B, npad, D = x.shape
    hd = D // H
    kern = functools.partial(_ab1_kernel, nreal, hd)
    return pl.pallas_call(
        kern,
        grid=(B,),
        in_specs=[
            pl.BlockSpec((1, npad, D), lambda i: (i, 0, 0)),
            pl.BlockSpec((1, npad, 1), lambda i: (i, 0, 0)),
            pl.BlockSpec((1, npad, 1), lambda i: (i, 0, 0)),
            pl.BlockSpec((1, D), lambda i: (0, 0)),
            pl.BlockSpec((1, D), lambda i: (0, 0)),
            pl.BlockSpec((D, D), lambda i: (0, 0)),
            pl.BlockSpec((D, D), lambda i: (0, 0)),
            pl.BlockSpec((D, D), lambda i: (0, 0)),
            pl.BlockSpec((D, D), lambda i: (0, 0)),
        ],
        out_specs=pl.BlockSpec((1, npad, D), lambda i: (i, 0, 0)),
        out_shape=jax.ShapeDtypeStruct((B, npad, D), jnp.float32),
        scratch_shapes=[pltpu.VMEM((npad, D), jnp.float32)],
    )(x, m, v, s.reshape(1, D), b.reshape(1, D), wq, wk, wv, wo)


def _ab2_kernel(nreal, K, hd, x_ref, m_ref, v_ref, s_ref, b_ref, wq_ref,
                wk_ref, wv_ref, wo_ref, o_ref, o_scr):
    x = x_ref[0]
    h = _ln_apply(x, m_ref[0], v_ref[0], s_ref[...], b_ref[...])
    k = jnp.dot(h, wk_ref[...], preferred_element_type=jnp.float32)
    v = jnp.dot(h, wv_ref[...], preferred_element_type=jnp.float32)
    q = jnp.dot(h[0:K, :], wq_ref[...], preferred_element_type=jnp.float32)
    o = _heads_attn(q, k, v, hd, nreal, o_scr)
    o_ref[0] = x[0:K, :] + jnp.dot(o, wo_ref[...],
                                   preferred_element_type=jnp.float32)


def _attn_block2(x, m, v, s, b, wq, wk, wv, wo, nreal, K):
    B, npad, D = x.shape
    hd = D // H
    kern = functools.partial(_ab2_kernel, nreal, K, hd)
    return pl.pallas_call(
        kern,
        grid=(B,),
        in_specs=[
            pl.BlockSpec((1, npad, D), lambda i: (i, 0, 0)),
            pl.BlockSpec((1, npad, 1), lambda i: (i, 0, 0)),
            pl.BlockSpec((1, npad, 1), lambda i: (i, 0, 0)),
            pl.BlockSpec((1, D), lambda i: (0, 0)),
            pl.BlockSpec((1, D), lambda i: (0, 0)),
            pl.BlockSpec((D, D), lambda i: (0, 0)),
            pl.BlockSpec((D, D), lambda i: (0, 0)),
            pl.BlockSpec((D, D), lambda i: (0, 0)),
            pl.BlockSpec((D, D), lambda i: (0, 0)),
        ],
        out_specs=pl.BlockSpec((1, K, D), lambda i: (i, 0, 0)),
        out_shape=jax.ShapeDtypeStruct((B, K, D), jnp.float32),
        scratch_shapes=[pltpu.VMEM((K, D), jnp.float32)],
    )(x, m, v, s.reshape(1, D), b.reshape(1, D), wq, wk, wv, wo)


def _mlp_kernel(x_ref, m_ref, v_ref, s_ref, b_ref, w1_ref, b1_ref, w2_ref,
                b2_ref, o_ref):
    x = x_ref[...]
    h = _ln_apply(x, m_ref[...], v_ref[...], s_ref[...], b_ref[...])
    u = jnp.dot(h, w1_ref[...], preferred_element_type=jnp.float32) + b1_ref[...]
    u = jax.nn.gelu(u)
    o_ref[...] = x + jnp.dot(u, w2_ref[...],
                             preferred_element_type=jnp.float32) + b2_ref[...]


def _mlp(x, m, v, s, b, w1, b1, w2, b2, bm=256):
    M, D = x.shape
    F = w1.shape[1]
    return pl.pallas_call(
        _mlp_kernel,
        grid=(M // bm,),
        in_specs=[
            pl.BlockSpec((bm, D), lambda i: (i, 0)),
            pl.BlockSpec((bm, 1), lambda i: (i, 0)),
            pl.BlockSpec((bm, 1), lambda i: (i, 0)),
            pl.BlockSpec((1, D), lambda i: (0, 0)),
            pl.BlockSpec((1, D), lambda i: (0, 0)),
            pl.BlockSpec((D, F), lambda i: (0, 0)),
            pl.BlockSpec((1, F), lambda i: (0, 0)),
            pl.BlockSpec((F, D), lambda i: (0, 0)),
            pl.BlockSpec((1, D), lambda i: (0, 0)),
        ],
        out_specs=pl.BlockSpec((bm, D), lambda i: (i, 0)),
        out_shape=jax.ShapeDtypeStruct((M, D), jnp.float32),
    )(x, m, v, s.reshape(1, D), b.reshape(1, D), w1, b1.reshape(1, F), w2,
      b2.reshape(1, D))


def _head_kernel(B, K, x_ref, w_ref, b_ref, g_ref, sel_ref, nr_ref):
    lat = jax.nn.sigmoid(
        jnp.dot(x_ref[...], w_ref[...], preferred_element_type=jnp.float32)
        + b_ref[...])
    lat3 = lat.reshape(B, K, lat.shape[-1])
    lb = lat3.astype(jnp.bfloat16).astype(jnp.float32)
    gb = g_ref[...].astype(jnp.bfloat16).astype(jnp.float32)
    aw = jnp.sum(lb * gb[:, None, :], axis=-1)
    aw = aw - jnp.max(aw, axis=1, keepdims=True)
    e = jnp.exp(aw)
    aw = e / jnp.sum(e, axis=1, keepdims=True)
    count = jnp.sum((aw > EPS).astype(jnp.int32), axis=1, keepdims=True)
    num_r = jnp.clip(count, 1, TOPK)
    ai = aw[:, :, None]
    aj = aw[:, None, :]
    ii = jax.lax.broadcasted_iota(jnp.int32, (B, K, K), 1)
    jj = jax.lax.broadcasted_iota(jnp.int32, (B, K, K), 2)
    cmp = (aj > ai) | ((aj == ai) & (jj < ii))
    rank = jnp.sum(cmp.astype(jnp.int32), axis=2)
    is_sel = rank < TOPK
    is_topr = rank < num_r
    prefix = jnp.sum((is_topr[:, None, :] & (jj < ii)).astype(jnp.int32),
                     axis=2)
    out_pos = jnp.where(is_topr, prefix, rank)
    bidx = jax.lax.broadcasted_iota(jnp.int32, (B, K), 0)
    g_out = jnp.where(is_sel, bidx * TOPK + out_pos, -1).reshape(1, B * K)
    rr = jax.lax.broadcasted_iota(jnp.int32, (B * TOPK, B * K), 0)
    onehot = (g_out == rr).astype(jnp.float32)
    sel = jnp.dot(onehot, lat, preferred_element_type=jnp.float32,
                  precision=jax.lax.Precision.HIGHEST)
    nrm = jnp.sqrt(jnp.sum(sel * sel, axis=-1, keepdims=True))
    sel_ref[...] = sel / jnp.maximum(nrm, 1e-12)
    nr_ref[...] = jnp.broadcast_to(num_r, (B, 128))


def _head(x, fc_w, fc_b, g, B, K):
    T = fc_w.shape[1]
    kern = functools.partial(_head_kernel, B, K)
    return pl.pallas_call(
        kern,
        out_shape=(jax.ShapeDtypeStruct((B * TOPK, T), jnp.float32),
                   jax.ShapeDtypeStruct((B, 128), jnp.int32)),
    )(x, fc_w, fc_b.reshape(1, T), g)


def kernel(img_global_feat, image_patch_feats, params):
    Bs, P, D = image_patch_feats.shape
    K = params["templates"].shape[1]
    n_real = K + P
    Np = ((n_real + 127) // 128) * 128

    tmpl = jnp.broadcast_to(params["templates"], (Bs, K, D))
    x0 = jnp.concatenate(
        [tmpl, image_patch_feats,
         jnp.zeros((Bs, Np - n_real, D), jnp.float32)], axis=1)

    p = params["layers"][0]
    m, v = _ln_stats(x0)
    x1 = _attn_block1(x0, m, v, p["ln1_s"], p["ln1_b"],
                      p["Wq"], p["Wk"], p["Wv"], p["Wo"], n_real)
    x1f = x1.reshape(Bs * Np, D)
    m, v = _ln_stats(x1f)
    x2 = _mlp(x1f, m, v, p["ln2_s"], p["ln2_b"], p["W1"],
              p["b1"], p["W2"], p["b2"])

    p = params["layers"][1]
    x2r = x2.reshape(Bs, Np, D)
    m, v = _ln_stats(x2r)
    x3 = _attn_block2(x2r, m, v, p["ln1_s"], p["ln1_b"], p["Wq"],
                      p["Wk"], p["Wv"], p["Wo"], n_real, K)
    x3f = x3.reshape(Bs * K, D)
    m, v = _ln_stats(x3f)
    x4 = _mlp(x3f, m, v, p["ln2_s"], p["ln2_b"], p["W1"],
              p["b1"], p["W2"], p["b2"])

    sel, nr = _head(x4, params["fc_W"], params["fc_b"], img_global_feat, Bs, K)
    T = params["fc_W"].shape[1]
    return sel.reshape(Bs, TOPK, T), nr[:, 0]
